# Initial kernel scaffold; baseline (speedup 1.0000x reference)
import jax
import jax.numpy as jnp
from jax.experimental import pallas as pl


def kernel(x, W, b):
    raise NotImplementedError("write your pallas kernel here")


# speedup vs baseline: 4.9207x; 4.9207x over previous
import functools

import jax
import jax.numpy as jnp
from jax.experimental import pallas as pl
from jax.experimental.pallas import tpu as pltpu

_TOPK = 8


def _gate_body(x_ref, w_ref, b_ref, g_ref, i_ref, s_ref):
    s = jnp.dot(x_ref[...], w_ref[...], preferred_element_type=jnp.float32)
    s = jax.nn.sigmoid(s + b_ref[...])
    s_ref[...] = s

    n_e = s.shape[1]
    lane = jax.lax.broadcasted_iota(jnp.int32, s.shape, 1)
    cur = s
    vals = []
    idxs = []
    for _ in range(_TOPK):
        m = jnp.max(cur, axis=1, keepdims=True)
        hit = cur == m
        idx = jnp.min(jnp.where(hit, lane, n_e), axis=1, keepdims=True)
        vals.append(m)
        idxs.append(idx)
        cur = jnp.where(lane == idx, -jnp.inf, cur)
    v = jnp.concatenate(vals, axis=1)
    g_ref[...] = v / jnp.sum(v, axis=1, keepdims=True)
    i_ref[...] = jnp.concatenate(idxs, axis=1)


@functools.partial(jax.jit, static_argnames=("block",))
def _gate(x, W, b, block=512):
    t, _ = x.shape
    n_e = W.shape[1]
    grid = (t // block,)
    return pl.pallas_call(
        _gate_body,
        grid=grid,
        in_specs=[
            pl.BlockSpec((block, x.shape[1]), lambda i: (i, 0)),
            pl.BlockSpec((W.shape[0], n_e), lambda i: (0, 0)),
            pl.BlockSpec((1, n_e), lambda i: (0, 0)),
        ],
        out_specs=[
            pl.BlockSpec((block, _TOPK), lambda i: (i, 0)),
            pl.BlockSpec((block, _TOPK), lambda i: (i, 0)),
            pl.BlockSpec((block, n_e), lambda i: (i, 0)),
        ],
        out_shape=[
            jax.ShapeDtypeStruct((t, _TOPK), jnp.float32),
            jax.ShapeDtypeStruct((t, _TOPK), jnp.int32),
            jax.ShapeDtypeStruct((t, n_e), jnp.float32),
        ],
        compiler_params=pltpu.CompilerParams(
            dimension_semantics=("arbitrary",),
        ),
    )(x, W, b.reshape(1, n_e))


def kernel(x, W, b):
    g_scores, indices, s = _gate(x, W, b)
    return (g_scores, indices, s)


# speedup vs baseline: 5.5650x; 1.1309x over previous
import functools

import jax
import jax.numpy as jnp
from jax.experimental import pallas as pl
from jax.experimental.pallas import tpu as pltpu

_TOPK = 8


def _gate_body(x_ref, w_ref, b_ref, g_ref, i_ref, s_ref):
    s = jnp.dot(x_ref[...], w_ref[...], preferred_element_type=jnp.float32)
    s = jax.nn.sigmoid(s + b_ref[...])
    s_ref[...] = s

    n_e = s.shape[1]
    lane = jax.lax.broadcasted_iota(jnp.int32, s.shape, 1)
    cur = s
    vals = []
    idxs = []
    for _ in range(_TOPK):
        m = jnp.max(cur, axis=1, keepdims=True)
        hit = cur == m
        idx = jnp.min(jnp.where(hit, lane, n_e), axis=1, keepdims=True)
        vals.append(m)
        idxs.append(idx)
        cur = jnp.where(lane == idx, -jnp.inf, cur)
    v = jnp.concatenate(vals, axis=1)
    g_ref[...] = v / jnp.sum(v, axis=1, keepdims=True)
    i_ref[...] = jnp.concatenate(idxs, axis=1)


@functools.partial(jax.jit, static_argnames=("block",))
def _gate(x, W, b, block=1024):
    t, _ = x.shape
    n_e = W.shape[1]
    grid = (t // block,)
    return pl.pallas_call(
        _gate_body,
        grid=grid,
        in_specs=[
            pl.BlockSpec((block, x.shape[1]), lambda i: (i, 0)),
            pl.BlockSpec((W.shape[0], n_e), lambda i: (0, 0)),
            pl.BlockSpec((1, n_e), lambda i: (0, 0)),
        ],
        out_specs=[
            pl.BlockSpec((block, _TOPK), lambda i: (i, 0)),
            pl.BlockSpec((block, _TOPK), lambda i: (i, 0)),
            pl.BlockSpec((block, n_e), lambda i: (i, 0)),
        ],
        out_shape=[
            jax.ShapeDtypeStruct((t, _TOPK), jnp.float32),
            jax.ShapeDtypeStruct((t, _TOPK), jnp.int32),
            jax.ShapeDtypeStruct((t, n_e), jnp.float32),
        ],
        compiler_params=pltpu.CompilerParams(
            dimension_semantics=("arbitrary",),
        ),
    )(x, W, b.reshape(1, n_e))


def kernel(x, W, b):
    g_scores, indices, s = _gate(x, W, b)
    return (g_scores, indices, s)


# speedup vs baseline: 7.9241x; 1.4239x over previous
import functools

import jax
import jax.numpy as jnp
from jax.experimental import pallas as pl
from jax.experimental.pallas import tpu as pltpu

_TOPK = 8


def _gate_body(x_ref, w_ref, b_ref, g_ref, i_ref, s_ref):
    st = jax.lax.dot_general(
        w_ref[...], x_ref[...], (((0,), (1,)), ((), ())),
        preferred_element_type=jnp.float32,
    )
    st = jax.nn.sigmoid(st + b_ref[...])
    s_ref[...] = st

    n_e = st.shape[0]
    expert = jax.lax.broadcasted_iota(jnp.int32, st.shape, 0)
    cur = st
    vals = []
    idxs = []
    for _ in range(_TOPK):
        m = jnp.max(cur, axis=0, keepdims=True)
        hit = cur == m
        idx = jnp.min(jnp.where(hit, expert, n_e), axis=0, keepdims=True)
        vals.append(m)
        idxs.append(idx)
        cur = jnp.where(expert == idx, -jnp.inf, cur)
    v = jnp.concatenate(vals, axis=0)
    g_ref[...] = v / jnp.sum(v, axis=0, keepdims=True)
    i_ref[...] = jnp.concatenate(idxs, axis=0)


@functools.partial(jax.jit, static_argnames=("block",))
def _gate(x, W, b, block=1024):
    t, d = x.shape
    n_e = W.shape[1]
    grid = (t // block,)
    gt, it, st = pl.pallas_call(
        _gate_body,
        grid=grid,
        in_specs=[
            pl.BlockSpec((block, d), lambda i: (i, 0)),
            pl.BlockSpec((d, n_e), lambda i: (0, 0)),
            pl.BlockSpec((n_e, 1), lambda i: (0, 0)),
        ],
        out_specs=[
            pl.BlockSpec((_TOPK, block), lambda i: (0, i)),
            pl.BlockSpec((_TOPK, block), lambda i: (0, i)),
            pl.BlockSpec((n_e, block), lambda i: (0, i)),
        ],
        out_shape=[
            jax.ShapeDtypeStruct((_TOPK, t), jnp.float32),
            jax.ShapeDtypeStruct((_TOPK, t), jnp.int32),
            jax.ShapeDtypeStruct((n_e, t), jnp.float32),
        ],
        compiler_params=pltpu.CompilerParams(
            dimension_semantics=("arbitrary",),
        ),
    )(x, W, b.reshape(n_e, 1))
    return gt.T, it.T, st.T


def kernel(x, W, b):
    g_scores, indices, s = _gate(x, W, b)
    return (g_scores, indices, s)
